# Initial kernel scaffold; baseline (speedup 1.0000x reference)
import jax
import jax.numpy as jnp
from jax.experimental import pallas as pl


def kernel(X_numerical, X_categorical, word_embeddings, num_embeddings):
    raise NotImplementedError("write your pallas kernel here")


# speedup vs baseline: 2.2715x; 2.2715x over previous
import functools

import numpy as np
import jax
import jax.numpy as jnp
from jax import lax
from jax.experimental import pallas as pl
from jax.experimental.pallas import tpu as pltpu
from jax.experimental.pallas import tpu_sc as plsc

B = 16384
N_NUM = 13
N_CAT = 26
H = 128
N_FIELDS = N_NUM + N_CAT

NC = 2
NS = 16
NW = NC * NS

CAT_PER_W = B * N_CAT // NW
CAT_CHUNK = 128
CAT_ITERS = CAT_PER_W // CAT_CHUNK

NB = 8
ROWS_PER_W = B // NW
NUM_ITERS = ROWS_PER_W // NB
NUM_CHUNK = NB * N_NUM

_p = np.arange(B * N_CAT, dtype=np.int64)
CAT_DST = ((_p // N_CAT) * N_FIELDS + N_NUM + (_p % N_CAT)).astype(np.int32)
_c = np.arange(B // NB, dtype=np.int64)
_k = np.arange(N_NUM, dtype=np.int64)
_b = np.arange(NB, dtype=np.int64)
NUM_DST = (
    ((_c[:, None, None] * NB + _b[None, None, :]) * N_FIELDS + _k[None, :, None])
    .reshape(-1)
    .astype(np.int32)
)


@functools.partial(
    pl.kernel,
    mesh=plsc.VectorSubcoreMesh(core_axis_name="c", subcore_axis_name="s"),
    out_type=jax.ShapeDtypeStruct((B * N_FIELDS, H), jnp.float32),
    scratch_types=[
        pltpu.VMEM((CAT_CHUNK,), jnp.int32),
        pltpu.VMEM((CAT_CHUNK,), jnp.int32),
        pltpu.VMEM((CAT_CHUNK, H), jnp.float32),
        pltpu.VMEM((N_NUM, H), jnp.float32),
        pltpu.VMEM((112,), jnp.float32),
        pltpu.VMEM((NUM_CHUNK,), jnp.int32),
        pltpu.VMEM((NUM_CHUNK, H), jnp.float32),
        pltpu.SemaphoreType.DMA,
    ],
)
def _embed_kernel(
    xnum_hbm, catidx_hbm, catdst_hbm, numdst_hbm, table_hbm, emb_hbm, out_hbm,
    idx_v, dst_v, rows_v, emb_v, xnum_v, numdst_v, numbuf_v, sem,
):
    wid = lax.axis_index("s") * NC + lax.axis_index("c")

    cat_base = wid * CAT_PER_W

    def cat_body(i, carry):
        off = pl.multiple_of(cat_base + i * CAT_CHUNK, CAT_CHUNK)
        pltpu.sync_copy(catidx_hbm.at[pl.ds(off, CAT_CHUNK)], idx_v)
        pltpu.sync_copy(catdst_hbm.at[pl.ds(off, CAT_CHUNK)], dst_v)
        pltpu.async_copy(table_hbm.at[idx_v], rows_v, sem).wait()
        pltpu.async_copy(rows_v, out_hbm.at[dst_v], sem).wait()
        return carry

    lax.fori_loop(0, CAT_ITERS, cat_body, 0, unroll=False)

    pltpu.sync_copy(emb_hbm, emb_v)
    chunk_base = wid * NUM_ITERS

    def num_body(c, carry):
        cp = chunk_base + c
        off = pl.multiple_of(cp * NUM_CHUNK, 8)
        pltpu.sync_copy(xnum_hbm.at[pl.ds(off, NUM_CHUNK)], xnum_v.at[pl.ds(0, NUM_CHUNK)])
        pltpu.sync_copy(numdst_hbm.at[pl.ds(off, NUM_CHUNK)], numdst_v)
        xv = [xnum_v[pl.ds(g * 16, 16)] for g in range(7)]
        for k in range(N_NUM):
            evecs = [emb_v[k, pl.ds(h * 16, 16)] for h in range(H // 16)]
            for b in range(NB):
                p = b * N_NUM + k
                x = xv[p // 16][p % 16]
                r = k * NB + b
                for h in range(H // 16):
                    numbuf_v[r, pl.ds(h * 16, 16)] = x * evecs[h]
        pltpu.async_copy(numbuf_v, out_hbm.at[numdst_v], sem).wait()
        return carry

    lax.fori_loop(0, NUM_ITERS, num_body, 0, unroll=False)


def kernel(X_numerical, X_categorical, word_embeddings, num_embeddings):
    xnum = X_numerical.reshape(-1)
    catidx = X_categorical.astype(jnp.int32).reshape(-1)
    emb = num_embeddings.reshape(N_NUM, H)
    out = _embed_kernel(
        xnum, catidx, jnp.asarray(CAT_DST), jnp.asarray(NUM_DST),
        word_embeddings, emb,
    )
    return out.reshape(B, N_FIELDS, H)


# speedup vs baseline: 2.9067x; 1.2796x over previous
import functools

import numpy as np
import jax
import jax.numpy as jnp
from jax import lax
from jax.experimental import pallas as pl
from jax.experimental.pallas import tpu as pltpu
from jax.experimental.pallas import tpu_sc as plsc

B = 16384
N_NUM = 13
N_CAT = 26
H = 128
N_FIELDS = N_NUM + N_CAT

NC = 2
NS = 16
NW = NC * NS

CAT_PER_W = B * N_CAT // NW
CAT_CHUNK = 128
CAT_ITERS = CAT_PER_W // CAT_CHUNK
NBUF = 4
CAT_GROUPS = CAT_ITERS // NBUF

NB = 8
ROWS_PER_W = B // NW
NUM_ITERS = ROWS_PER_W // NB
NUM_CHUNK = NB * N_NUM
NUM_GROUPS = NUM_ITERS // 2

_p = np.arange(B * N_CAT, dtype=np.int64)
CAT_DST = (
    ((_p // N_CAT) * N_FIELDS + N_NUM + (_p % N_CAT))
    .astype(np.int32)
    .reshape(-1, CAT_CHUNK)
)
_c = np.arange(B // NB, dtype=np.int64)
_k = np.arange(N_NUM, dtype=np.int64)
_b = np.arange(NB, dtype=np.int64)
NUM_DST = (
    ((_c[:, None, None] * NB + _b[None, None, :]) * N_FIELDS + _k[None, :, None])
    .reshape(-1, NUM_CHUNK)
    .astype(np.int32)
)


@functools.partial(
    pl.kernel,
    mesh=plsc.VectorSubcoreMesh(core_axis_name="c", subcore_axis_name="s"),
    out_type=jax.ShapeDtypeStruct((B * N_FIELDS, H), jnp.float32),
    scratch_types=[
        pltpu.VMEM((CAT_ITERS, CAT_CHUNK), jnp.int32),
        pltpu.VMEM((CAT_ITERS, CAT_CHUNK), jnp.int32),
        pltpu.VMEM((CAT_CHUNK, H), jnp.float32),
        pltpu.VMEM((CAT_CHUNK, H), jnp.float32),
        pltpu.VMEM((CAT_CHUNK, H), jnp.float32),
        pltpu.VMEM((CAT_CHUNK, H), jnp.float32),
        pltpu.VMEM((N_NUM, H), jnp.float32),
        pltpu.VMEM((ROWS_PER_W * N_NUM + 16,), jnp.float32),
        pltpu.VMEM((NUM_ITERS, NUM_CHUNK), jnp.int32),
        pltpu.SemaphoreType.DMA,
        pltpu.SemaphoreType.DMA,
        pltpu.SemaphoreType.DMA,
        pltpu.SemaphoreType.DMA,
        pltpu.SemaphoreType.DMA,
        pltpu.SemaphoreType.DMA,
        pltpu.SemaphoreType.DMA,
        pltpu.SemaphoreType.DMA,
    ],
)
def _embed_kernel(
    xnum_hbm, catidx_hbm, catdst_hbm, numdst_hbm, table_hbm, emb_hbm, out_hbm,
    idx_v, dst_v, buf0, buf1, buf2, buf3, emb_v, xnum_v, numdst_v,
    g0, g1, g2, g3, s0, s1, s2, s3,
):
    bufs = (buf0, buf1, buf2, buf3)
    gsem = (g0, g1, g2, g3)
    ssem = (s0, s1, s2, s3)
    wid = lax.axis_index("s") * NC + lax.axis_index("c")

    pltpu.sync_copy(catidx_hbm.at[pl.ds(wid * CAT_ITERS, CAT_ITERS)], idx_v)
    pltpu.sync_copy(catdst_hbm.at[pl.ds(wid * CAT_ITERS, CAT_ITERS)], dst_v)
    pltpu.sync_copy(emb_hbm, emb_v)
    pltpu.sync_copy(
        xnum_hbm.at[pl.ds(wid * ROWS_PER_W * N_NUM, ROWS_PER_W * N_NUM)],
        xnum_v.at[pl.ds(0, ROWS_PER_W * N_NUM)],
    )
    pltpu.sync_copy(numdst_hbm.at[pl.ds(wid * NUM_ITERS, NUM_ITERS)], numdst_v)

    for b in range(NBUF):
        pltpu.async_copy(table_hbm.at[idx_v.at[b]], bufs[b], gsem[b])

    def cat_body(g, carry):
        base = g * NBUF
        for b in range(NBUF):
            pltpu.make_async_copy(table_hbm.at[idx_v.at[b]], bufs[b], gsem[b]).wait()
            pltpu.async_copy(bufs[b], out_hbm.at[dst_v.at[base + b]], ssem[b])
        for b in range(NBUF):
            pltpu.make_async_copy(bufs[b], out_hbm.at[dst_v.at[b]], ssem[b]).wait()

            @pl.when(g < CAT_GROUPS - 1)
            def _():
                pltpu.async_copy(
                    table_hbm.at[idx_v.at[base + NBUF + b]], bufs[b], gsem[b]
                )

        return carry

    lax.fori_loop(0, CAT_GROUPS, cat_body, 0, unroll=False)

    def num_body(g, carry):
        for par in range(2):
            c = g * 2 + par
            coff = c * NUM_CHUNK

            @pl.when(g >= 1)
            def _():
                pltpu.make_async_copy(
                    bufs[par].at[pl.ds(0, NUM_CHUNK)],
                    out_hbm.at[numdst_v.at[0]],
                    ssem[par],
                ).wait()

            xv = [xnum_v[pl.ds(coff + i * 16, 16)] for i in range(7)]
            for k in range(N_NUM):
                evecs = [emb_v[k, pl.ds(h * 16, 16)] for h in range(H // 16)]
                for bb in range(NB):
                    p = bb * N_NUM + k
                    x = xv[p // 16][p % 16]
                    r = k * NB + bb
                    for h in range(H // 16):
                        bufs[par][r, pl.ds(h * 16, 16)] = x * evecs[h]
            pltpu.async_copy(
                bufs[par].at[pl.ds(0, NUM_CHUNK)],
                out_hbm.at[numdst_v.at[c]],
                ssem[par],
            )
        return carry

    lax.fori_loop(0, NUM_GROUPS, num_body, 0, unroll=False)
    for par in range(2):
        pltpu.make_async_copy(
            bufs[par].at[pl.ds(0, NUM_CHUNK)], out_hbm.at[numdst_v.at[0]], ssem[par]
        ).wait()


def kernel(X_numerical, X_categorical, word_embeddings, num_embeddings):
    xnum = X_numerical.reshape(-1)
    catidx = X_categorical.astype(jnp.int32).reshape(-1, CAT_CHUNK)
    emb = num_embeddings.reshape(N_NUM, H)
    out = _embed_kernel(
        xnum, catidx, jnp.asarray(CAT_DST), jnp.asarray(NUM_DST),
        word_embeddings, emb,
    )
    return out.reshape(B, N_FIELDS, H)


# speedup vs baseline: 9.1551x; 3.1497x over previous
import functools

import jax
import jax.numpy as jnp
from jax import lax
from jax.experimental import pallas as pl
from jax.experimental.pallas import tpu as pltpu
from jax.experimental.pallas import tpu_sc as plsc

B = 16384
N_NUM = 13
N_CAT = 26
H = 128
N_FIELDS = N_NUM + N_CAT

NC = 2
NS = 16
NW = NC * NS

ROWS_PER_W = B // NW
CHUNK = 128
NCH = ROWS_PER_W // CHUNK
NBUF = 4


@functools.partial(
    pl.kernel,
    mesh=plsc.VectorSubcoreMesh(core_axis_name="c", subcore_axis_name="s"),
    out_type=jax.ShapeDtypeStruct((B * N_FIELDS, H), jnp.float32),
    scratch_types=[
        pltpu.VMEM((N_CAT * ROWS_PER_W,), jnp.int32),
        pltpu.VMEM((N_NUM * ROWS_PER_W + 16,), jnp.float32),
        pltpu.VMEM((N_NUM * H,), jnp.float32),
        pltpu.VMEM((CHUNK, H), jnp.float32),
        pltpu.VMEM((CHUNK, H), jnp.float32),
        pltpu.VMEM((CHUNK, H), jnp.float32),
        pltpu.VMEM((CHUNK, H), jnp.float32),
        pltpu.SemaphoreType.DMA,
        pltpu.SemaphoreType.DMA,
        pltpu.SemaphoreType.DMA,
        pltpu.SemaphoreType.DMA,
        pltpu.SemaphoreType.DMA,
        pltpu.SemaphoreType.DMA,
        pltpu.SemaphoreType.DMA,
        pltpu.SemaphoreType.DMA,
    ],
)
def _embed_kernel(
    xnum_hbm, catidx_hbm, table_hbm, emb_hbm, out_hbm,
    idx_v, xnum_v, emb_v, buf0, buf1, buf2, buf3,
    g0, g1, g2, g3, s0, s1, s2, s3,
):
    bufs = (buf0, buf1, buf2, buf3)
    gsem = (g0, g1, g2, g3)
    ssem = (s0, s1, s2, s3)
    wid = lax.axis_index("s") * NC + lax.axis_index("c")
    wb = wid * ROWS_PER_W

    for j in range(N_CAT):
        pltpu.async_copy(
            catidx_hbm.at[pl.ds(j * B + wb, ROWS_PER_W)],
            idx_v.at[pl.ds(j * ROWS_PER_W, ROWS_PER_W)],
            g0,
        )
    for k in range(N_NUM):
        pltpu.async_copy(
            xnum_hbm.at[pl.ds(k * B + wb, ROWS_PER_W)],
            xnum_v.at[pl.ds(k * ROWS_PER_W, ROWS_PER_W)],
            g1,
        )
    pltpu.async_copy(emb_hbm, emb_v, g2)
    for j in range(N_CAT):
        pltpu.make_async_copy(
            catidx_hbm.at[pl.ds(0, ROWS_PER_W)], idx_v.at[pl.ds(0, ROWS_PER_W)], g0
        ).wait()
    for k in range(N_NUM):
        pltpu.make_async_copy(
            xnum_hbm.at[pl.ds(0, ROWS_PER_W)], xnum_v.at[pl.ds(0, ROWS_PER_W)], g1
        ).wait()
    pltpu.make_async_copy(emb_hbm, emb_v, g2).wait()

    def idx_ref(j, b):
        return idx_v.at[pl.ds(j * ROWS_PER_W + b * CHUNK, CHUNK)]

    for b in range(NBUF):
        pltpu.async_copy(table_hbm.at[idx_ref(0, b)], bufs[b], gsem[b])

    def cat_body(j, carry):
        out_base = (N_NUM + j) * B + wb
        for b in range(NBUF):
            pltpu.make_async_copy(table_hbm.at[idx_ref(0, b)], bufs[b], gsem[b]).wait()
            pltpu.async_copy(
                bufs[b], out_hbm.at[pl.ds(out_base + b * CHUNK, CHUNK)], ssem[b]
            )
        for b in range(NBUF):
            pltpu.make_async_copy(
                bufs[b], out_hbm.at[pl.ds(0, CHUNK)], ssem[b]
            ).wait()

            @pl.when(j < N_CAT - 1)
            def _():
                pltpu.async_copy(table_hbm.at[idx_ref(j + 1, b)], bufs[b], gsem[b])

        return carry

    lax.fori_loop(0, N_CAT, cat_body, 0, unroll=False)

    NUM_CHUNKS = N_NUM * NCH

    def num_body(g, carry):
        for par in range(2):
            c = g * 2 + par
            k = c // NCH
            bo = c % NCH

            @pl.when(g >= 1)
            def _():
                pltpu.make_async_copy(
                    bufs[par], out_hbm.at[pl.ds(0, CHUNK)], ssem[par]
                ).wait()

            base = k * ROWS_PER_W + bo * CHUNK
            evecs = [emb_v[pl.ds(k * H + h * 16, 16)] for h in range(H // 16)]
            for i in range(CHUNK // 16):
                xvec = xnum_v[pl.ds(base + i * 16, 16)]
                for l in range(16):
                    x = xvec[l]
                    r = i * 16 + l
                    for h in range(H // 16):
                        bufs[par][r, pl.ds(h * 16, 16)] = x * evecs[h]
            pltpu.async_copy(
                bufs[par], out_hbm.at[pl.ds(k * B + wb + bo * CHUNK, CHUNK)], ssem[par]
            )
        return carry

    lax.fori_loop(0, NUM_CHUNKS // 2, num_body, 0, unroll=False)
    for par in range(2):
        pltpu.make_async_copy(bufs[par], out_hbm.at[pl.ds(0, CHUNK)], ssem[par]).wait()


def kernel(X_numerical, X_categorical, word_embeddings, num_embeddings):
    xnum = jnp.transpose(X_numerical).reshape(-1)
    catidx = jnp.transpose(X_categorical.astype(jnp.int32)).reshape(-1)
    emb = num_embeddings.reshape(-1)
    out = _embed_kernel(xnum, catidx, word_embeddings, emb)
    out = out.reshape(N_FIELDS, B, H)
    return jnp.transpose(out, (1, 0, 2))


# speedup vs baseline: 10.4875x; 1.1455x over previous
import functools

import jax
import jax.numpy as jnp
from jax import lax
from jax.experimental import pallas as pl
from jax.experimental.pallas import tpu as pltpu
from jax.experimental.pallas import tpu_sc as plsc

B = 16384
N_NUM = 13
N_CAT = 26
H = 128
N_FIELDS = N_NUM + N_CAT

NC = 2
NS = 16
NW = NC * NS

ROWS_PER_W = B // NW
CHUNK = 128
NCH = ROWS_PER_W // CHUNK
NBUF = 4


@functools.partial(
    pl.kernel,
    mesh=plsc.VectorSubcoreMesh(core_axis_name="c", subcore_axis_name="s"),
    out_type=jax.ShapeDtypeStruct((B * N_FIELDS, H), jnp.float32),
    scratch_types=[
        pltpu.VMEM((N_CAT * ROWS_PER_W,), jnp.int32),
        pltpu.VMEM((N_NUM * ROWS_PER_W + 16,), jnp.float32),
        pltpu.VMEM((N_NUM * H,), jnp.float32),
        pltpu.VMEM((CHUNK, H), jnp.float32),
        pltpu.VMEM((CHUNK, H), jnp.float32),
        pltpu.VMEM((CHUNK, H), jnp.float32),
        pltpu.VMEM((CHUNK, H), jnp.float32),
        pltpu.VMEM((CHUNK, H), jnp.float32),
        pltpu.VMEM((CHUNK, H), jnp.float32),
        pltpu.SemaphoreType.DMA,
        pltpu.SemaphoreType.DMA,
        pltpu.SemaphoreType.DMA,
        pltpu.SemaphoreType.DMA,
        pltpu.SemaphoreType.DMA,
        pltpu.SemaphoreType.DMA,
        pltpu.SemaphoreType.DMA,
        pltpu.SemaphoreType.DMA,
        pltpu.SemaphoreType.DMA,
        pltpu.SemaphoreType.DMA,
    ],
)
def _embed_kernel(
    xnum_hbm, catidx_hbm, table_hbm, emb_hbm, out_hbm,
    idx_v, xnum_v, emb_v, buf0, buf1, buf2, buf3, nbuf0, nbuf1,
    g0, g1, g2, g3, s0, s1, s2, s3, n0, n1,
):
    bufs = (buf0, buf1, buf2, buf3)
    nbufs = (nbuf0, nbuf1)
    gsem = (g0, g1, g2, g3)
    ssem = (s0, s1, s2, s3)
    nsem = (n0, n1)
    wid = lax.axis_index("s") * NC + lax.axis_index("c")
    wb = wid * ROWS_PER_W

    for j in range(N_CAT):
        pltpu.async_copy(
            catidx_hbm.at[pl.ds(j * B + wb, ROWS_PER_W)],
            idx_v.at[pl.ds(j * ROWS_PER_W, ROWS_PER_W)],
            g0,
        )
    for k in range(N_NUM):
        pltpu.async_copy(
            xnum_hbm.at[pl.ds(k * B + wb, ROWS_PER_W)],
            xnum_v.at[pl.ds(k * ROWS_PER_W, ROWS_PER_W)],
            g1,
        )
    pltpu.async_copy(emb_hbm, emb_v, g2)
    for j in range(N_CAT):
        pltpu.make_async_copy(
            catidx_hbm.at[pl.ds(0, ROWS_PER_W)], idx_v.at[pl.ds(0, ROWS_PER_W)], g0
        ).wait()
    for k in range(N_NUM):
        pltpu.make_async_copy(
            xnum_hbm.at[pl.ds(0, ROWS_PER_W)], xnum_v.at[pl.ds(0, ROWS_PER_W)], g1
        ).wait()
    pltpu.make_async_copy(emb_hbm, emb_v, g2).wait()

    def idx_ref(j, b):
        return idx_v.at[pl.ds(j * ROWS_PER_W + b * CHUNK, CHUNK)]

    for b in range(NBUF):
        pltpu.async_copy(table_hbm.at[idx_ref(0, b)], bufs[b], gsem[b])

    def main_body(j, carry):
        out_base = (N_NUM + j) * B + wb
        for b in range(NBUF):
            pltpu.make_async_copy(table_hbm.at[idx_ref(0, b)], bufs[b], gsem[b]).wait()
            pltpu.async_copy(
                bufs[b], out_hbm.at[pl.ds(out_base + b * CHUNK, CHUNK)], ssem[b]
            )
        for par in range(2):
            c = j * 2 + par
            k = c // NCH
            bo = c % NCH

            @pl.when(j >= 1)
            def _():
                pltpu.make_async_copy(
                    nbufs[par], out_hbm.at[pl.ds(0, CHUNK)], nsem[par]
                ).wait()

            base = k * ROWS_PER_W + bo * CHUNK
            evecs = [emb_v[pl.ds(k * H + h * 16, 16)] for h in range(H // 16)]
            for i in range(CHUNK // 16):
                xvec = xnum_v[pl.ds(base + i * 16, 16)]
                for l in range(16):
                    x = xvec[l]
                    r = i * 16 + l
                    for h in range(H // 16):
                        nbufs[par][r, pl.ds(h * 16, 16)] = x * evecs[h]
            pltpu.async_copy(
                nbufs[par], out_hbm.at[pl.ds(k * B + wb + bo * CHUNK, CHUNK)], nsem[par]
            )
        for b in range(NBUF):
            pltpu.make_async_copy(
                bufs[b], out_hbm.at[pl.ds(0, CHUNK)], ssem[b]
            ).wait()

            @pl.when(j < N_CAT - 1)
            def _():
                pltpu.async_copy(table_hbm.at[idx_ref(j + 1, b)], bufs[b], gsem[b])

        return carry

    lax.fori_loop(0, N_CAT, main_body, 0, unroll=False)
    for par in range(2):
        pltpu.make_async_copy(nbufs[par], out_hbm.at[pl.ds(0, CHUNK)], nsem[par]).wait()


def kernel(X_numerical, X_categorical, word_embeddings, num_embeddings):
    xnum = jnp.transpose(X_numerical).reshape(-1)
    catidx = jnp.transpose(X_categorical.astype(jnp.int32)).reshape(-1)
    emb = num_embeddings.reshape(-1)
    out = _embed_kernel(xnum, catidx, word_embeddings, emb)
    out = out.reshape(N_FIELDS, B, H)
    return jnp.transpose(out, (1, 0, 2))


# speedup vs baseline: 10.6144x; 1.0121x over previous
import functools

import jax
import jax.numpy as jnp
from jax import lax
from jax.experimental import pallas as pl
from jax.experimental.pallas import tpu as pltpu
from jax.experimental.pallas import tpu_sc as plsc

B = 16384
N_NUM = 13
N_CAT = 26
H = 128
N_FIELDS = N_NUM + N_CAT

NC = 2
NS = 16
NW = NC * NS

ROWS_PER_W = B // NW
CHUNK = 128
NCH = ROWS_PER_W // CHUNK
NBUF = 4


@functools.partial(
    pl.kernel,
    mesh=plsc.VectorSubcoreMesh(core_axis_name="c", subcore_axis_name="s"),
    out_type=jax.ShapeDtypeStruct((B * N_FIELDS, H), jnp.float32),
    scratch_types=[
        pltpu.VMEM((N_CAT * ROWS_PER_W,), jnp.int32),
        pltpu.VMEM((N_NUM * ROWS_PER_W + 16,), jnp.float32),
        pltpu.VMEM((N_NUM * H,), jnp.float32),
        pltpu.VMEM((CHUNK, H), jnp.float32),
        pltpu.VMEM((CHUNK, H), jnp.float32),
        pltpu.VMEM((CHUNK, H), jnp.float32),
        pltpu.VMEM((CHUNK, H), jnp.float32),
        pltpu.VMEM((CHUNK, H), jnp.float32),
        pltpu.VMEM((CHUNK, H), jnp.float32),
        pltpu.SemaphoreType.DMA,
        pltpu.SemaphoreType.DMA,
        pltpu.SemaphoreType.DMA,
        pltpu.SemaphoreType.DMA,
        pltpu.SemaphoreType.DMA,
        pltpu.SemaphoreType.DMA,
        pltpu.SemaphoreType.DMA,
        pltpu.SemaphoreType.DMA,
        pltpu.SemaphoreType.DMA,
        pltpu.SemaphoreType.DMA,
    ],
)
def _embed_kernel(
    xnum_hbm, catidx_hbm, table_hbm, emb_hbm, out_hbm,
    idx_v, xnum_v, emb_v, buf0, buf1, buf2, buf3, nbuf0, nbuf1,
    g0, g1, g2, g3, s0, s1, s2, s3, n0, n1,
):
    bufs = (buf0, buf1, buf2, buf3)
    nbufs = (nbuf0, nbuf1)
    gsem = (g0, g1, g2, g3)
    ssem = (s0, s1, s2, s3)
    nsem = (n0, n1)
    wid = lax.axis_index("s") * NC + lax.axis_index("c")
    wb = wid * ROWS_PER_W

    def idx_ref(j, b):
        return idx_v.at[pl.ds(j * ROWS_PER_W + b * CHUNK, CHUNK)]

    pltpu.async_copy(
        catidx_hbm.at[pl.ds(wb, ROWS_PER_W)], idx_v.at[pl.ds(0, ROWS_PER_W)], g0
    )
    for j in range(1, N_CAT):
        pltpu.async_copy(
            catidx_hbm.at[pl.ds(j * B + wb, ROWS_PER_W)],
            idx_v.at[pl.ds(j * ROWS_PER_W, ROWS_PER_W)],
            s0,
        )
    for k in range(N_NUM):
        pltpu.async_copy(
            xnum_hbm.at[pl.ds(k * B + wb, ROWS_PER_W)],
            xnum_v.at[pl.ds(k * ROWS_PER_W, ROWS_PER_W)],
            s1,
        )
    pltpu.async_copy(emb_hbm, emb_v, s2)
    pltpu.make_async_copy(
        catidx_hbm.at[pl.ds(0, ROWS_PER_W)], idx_v.at[pl.ds(0, ROWS_PER_W)], g0
    ).wait()
    for b in range(NBUF):
        pltpu.async_copy(table_hbm.at[idx_ref(0, b)], bufs[b], gsem[b])
    for j in range(1, N_CAT):
        pltpu.make_async_copy(
            catidx_hbm.at[pl.ds(0, ROWS_PER_W)], idx_v.at[pl.ds(0, ROWS_PER_W)], s0
        ).wait()
    for k in range(N_NUM):
        pltpu.make_async_copy(
            xnum_hbm.at[pl.ds(0, ROWS_PER_W)], xnum_v.at[pl.ds(0, ROWS_PER_W)], s1
        ).wait()
    pltpu.make_async_copy(emb_hbm, emb_v, s2).wait()

    def num_chunk(j, par):
        c = j * 2 + par
        k = c // NCH
        bo = c % NCH

        @pl.when(j >= 1)
        def _():
            pltpu.make_async_copy(
                nbufs[par], out_hbm.at[pl.ds(0, CHUNK)], nsem[par]
            ).wait()

        base = k * ROWS_PER_W + bo * CHUNK
        evecs = [emb_v[pl.ds(k * H + h * 16, 16)] for h in range(H // 16)]
        for i in range(CHUNK // 16):
            xvec = xnum_v[pl.ds(base + i * 16, 16)]
            for l in range(16):
                x = xvec[l]
                r = i * 16 + l
                for h in range(H // 16):
                    nbufs[par][r, pl.ds(h * 16, 16)] = x * evecs[h]
        pltpu.async_copy(
            nbufs[par], out_hbm.at[pl.ds(k * B + wb + bo * CHUNK, CHUNK)], nsem[par]
        )

    def drain_and_refill(j, b):
        pltpu.make_async_copy(bufs[b], out_hbm.at[pl.ds(0, CHUNK)], ssem[b]).wait()

        @pl.when(j < N_CAT - 1)
        def _():
            pltpu.async_copy(table_hbm.at[idx_ref(j + 1, b)], bufs[b], gsem[b])

    def main_body(j, carry):
        out_base = (N_NUM + j) * B + wb
        for b in range(NBUF):
            pltpu.make_async_copy(table_hbm.at[idx_ref(0, b)], bufs[b], gsem[b]).wait()
            pltpu.async_copy(
                bufs[b], out_hbm.at[pl.ds(out_base + b * CHUNK, CHUNK)], ssem[b]
            )
        num_chunk(j, 0)
        for b in (0, 1):
            drain_and_refill(j, b)
        num_chunk(j, 1)
        for b in (2, 3):
            drain_and_refill(j, b)
        return carry

    lax.fori_loop(0, N_CAT, main_body, 0, unroll=False)
    for par in range(2):
        pltpu.make_async_copy(nbufs[par], out_hbm.at[pl.ds(0, CHUNK)], nsem[par]).wait()


def kernel(X_numerical, X_categorical, word_embeddings, num_embeddings):
    xnum = jnp.transpose(X_numerical).reshape(-1)
    catidx = jnp.transpose(X_categorical.astype(jnp.int32)).reshape(-1)
    emb = num_embeddings.reshape(-1)
    out = _embed_kernel(xnum, catidx, word_embeddings, emb)
    out = out.reshape(N_FIELDS, B, H)
    return jnp.transpose(out, (1, 0, 2))
